# baseline (speedup 1.0000x reference)
import jax
import jax.numpy as jnp
from jax.experimental import pallas as pl
from jax.experimental.pallas import tpu as pltpu

ALPHA = 0.25
GAMMA = 2.0
COST_CLASS = 2.0
COST_BBOX = 5.0
COST_GIOU = 2.0
COST_DENSITY = 1.0


def _matcher_body(logits_ref, pb_ref, lab_ref, tb_ref, dm_ref,
                  match_ref, c_ref):
    b = pl.program_id(0)
    Q, NC = logits_ref.shape[1], logits_ref.shape[2]
    G = lab_ref.shape[1]
    H, W = dm_ref.shape[1], dm_ref.shape[2]
    tbT0 = tb_ref[0].T

    logits = logits_ref[0]
    p = jax.nn.sigmoid(logits)
    one_m_p = 1.0 - p
    pos = ALPHA * (one_m_p * one_m_p) * (-jnp.log(p + 1e-08))
    neg = (1.0 - ALPHA) * (p * p) * (-jnp.log(1.0 - p + 1e-08))
    diff = pos - neg
    class_iota = jax.lax.broadcasted_iota(jnp.int32, (NC, G), 0)
    onehot = (class_iota == lab_ref[0:1, :]).astype(jnp.float32)
    c_class = jnp.dot(diff, onehot, preferred_element_type=jnp.float32,
                      precision=jax.lax.Precision.HIGHEST)

    pb = pb_ref[0]
    cx, cy = pb[:, 0:1], pb[:, 1:2]
    w, h = pb[:, 2:3], pb[:, 3:4]
    t_cx, t_cy = tbT0[0:1, :], tbT0[1:2, :]
    t_w, t_h = tbT0[2:3, :], tbT0[3:4, :]
    c_bbox = (jnp.abs(cx - t_cx) + jnp.abs(cy - t_cy)
              + jnp.abs(w - t_w) + jnp.abs(h - t_h))

    hw, hh = 0.5 * w, 0.5 * h
    x0, y0 = cx - hw, cy - hh
    x1, y1 = cx + hw, cy + hh
    thw, thh = 0.5 * t_w, 0.5 * t_h
    tx0, ty0 = t_cx - thw, t_cy - thh
    tx1, ty1 = t_cx + thw, t_cy + thh
    area1 = (x1 - x0) * (y1 - y0)
    area2 = (tx1 - tx0) * (ty1 - ty0)
    iw = jnp.clip(jnp.minimum(x1, tx1) - jnp.maximum(x0, tx0), 0.0)
    ih = jnp.clip(jnp.minimum(y1, ty1) - jnp.maximum(y0, ty0), 0.0)
    inter = iw * ih
    union = area1 + area2 - inter
    iou = inter / union
    ew = jnp.maximum(x1, tx1) - jnp.minimum(x0, tx0)
    eh = jnp.maximum(y1, ty1) - jnp.minimum(y0, ty0)
    earea = ew * eh
    c_giou = -(iou - (earea - union) / earea)

    img = dm_ref[0]
    dn = img / (jnp.max(img) + 1e-06)
    gx = cx * 2.0 - 1.0
    gy = cy * 2.0 - 1.0
    ix = ((gx + 1.0) * W - 1.0) / 2.0
    iy = ((gy + 1.0) * H - 1.0) / 2.0
    col_iota = jax.lax.broadcasted_iota(jnp.int32, (Q, W), 1).astype(jnp.float32)
    wmat_y = jnp.maximum(1.0 - jnp.abs(col_iota - iy), 0.0)
    wmat_x = jnp.maximum(1.0 - jnp.abs(col_iota - ix), 0.0)
    rows = jnp.dot(wmat_y, dn, preferred_element_type=jnp.float32,
                   precision=jax.lax.Precision.HIGHEST)
    sampled = jnp.sum(rows * wmat_x, axis=1, keepdims=True)

    C = (COST_BBOX * c_bbox + COST_CLASS * c_class
         + COST_GIOU * c_giou + COST_DENSITY * (-sampled))

    areas = (tb_ref[b][:, 2:3] * tb_ref[b][:, 3:4]).T
    kvec = jnp.where(areas < 0.005, 6, 2)
    q_iota = jax.lax.broadcasted_iota(jnp.int32, (Q, G), 0)
    Cw = C
    mm = jnp.zeros((Q, G), dtype=jnp.float32)
    for s in range(6):
        mval = jnp.min(Cw, axis=0, keepdims=True)
        idx = jnp.min(jnp.where(Cw == mval, q_iota, Q), axis=0,
                      keepdims=True)
        sel = q_iota == idx
        mm = jnp.maximum(mm, jnp.where(jnp.logical_and(sel, s < kvec),
                                       1.0, 0.0))
        Cw = jnp.where(sel, jnp.inf, Cw)

    nmatch = jnp.sum(mm, axis=1, keepdims=True)
    conflict = nmatch > 1.5
    matched = mm > 0.5
    masked_C = jnp.where(matched, C, jnp.inf)
    mrow = jnp.min(masked_C, axis=1, keepdims=True)
    g_iota = jax.lax.broadcasted_iota(jnp.int32, (Q, G), 1)
    bidx = jnp.min(jnp.where(masked_C == mrow, g_iota, G), axis=1,
                   keepdims=True)
    mm = jnp.where(conflict, jnp.where(g_iota == bidx, 1.0, 0.0), mm)

    match_ref[0] = mm > 0.5
    c_ref[0] = C


@jax.jit
def kernel(pred_logits, pred_boxes, tgt_labels, tgt_boxes, density_map):
    B, Q, NC = pred_logits.shape
    G = tgt_labels.shape[1]
    H, W = density_map.shape[2], density_map.shape[3]

    dm = density_map.reshape(B, H, W)

    matching, c_all = pl.pallas_call(
        _matcher_body,
        grid=(B,),
        in_specs=[
            pl.BlockSpec((1, Q, NC), lambda b: (b, 0, 0)),
            pl.BlockSpec((1, Q, 4), lambda b: (b, 0, 0)),
            pl.BlockSpec((B, G), lambda b: (0, 0)),
            pl.BlockSpec((B, G, 4), lambda b: (0, 0, 0)),
            pl.BlockSpec((1, H, W), lambda b: (b, 0, 0)),
        ],
        out_specs=[
            pl.BlockSpec((1, Q, G), lambda b: (b, 0, 0)),
            pl.BlockSpec((1, Q, G), lambda b: (b, 0, 0)),
        ],
        out_shape=[
            jax.ShapeDtypeStruct((B, Q, G), jnp.bool_),
            jax.ShapeDtypeStruct((B, Q, G), jnp.float32),
        ],
        compiler_params=pltpu.CompilerParams(
            dimension_semantics=("parallel",),
        ),
    )(pred_logits, pred_boxes, tgt_labels, tgt_boxes, dm)
    return matching, c_all


# speedup vs baseline: 1.0618x; 1.0618x over previous
import jax
import jax.numpy as jnp
from jax.experimental import pallas as pl
from jax.experimental.pallas import tpu as pltpu

ALPHA = 0.25
GAMMA = 2.0
COST_CLASS = 2.0
COST_BBOX = 5.0
COST_GIOU = 2.0
COST_DENSITY = 1.0


def _matcher_body(logits_ref, pb_ref, lab_ref, tbT_ref, dm_ref,
                  match_ref, c_ref):
    b = pl.program_id(0)
    Q, NC = logits_ref.shape[1], logits_ref.shape[2]
    G = lab_ref.shape[1]
    H, W = dm_ref.shape[1], dm_ref.shape[2]

    logits = logits_ref[0]
    p = jax.nn.sigmoid(logits)
    one_m_p = 1.0 - p
    pos = ALPHA * (one_m_p * one_m_p) * (-jnp.log(p + 1e-08))
    neg = (1.0 - ALPHA) * (p * p) * (-jnp.log(1.0 - p + 1e-08))
    diff = pos - neg
    class_iota = jax.lax.broadcasted_iota(jnp.int32, (NC, G), 0)
    onehot = (class_iota == lab_ref[0:1, :]).astype(jnp.float32)
    c_class = jnp.dot(diff, onehot, preferred_element_type=jnp.float32,
                      precision=jax.lax.Precision.HIGHEST)

    pb = pb_ref[0]
    cx, cy = pb[:, 0:1], pb[:, 1:2]
    w, h = pb[:, 2:3], pb[:, 3:4]
    t_cx, t_cy = tbT_ref[0, 0:1, :], tbT_ref[0, 1:2, :]
    t_w, t_h = tbT_ref[0, 2:3, :], tbT_ref[0, 3:4, :]
    c_bbox = (jnp.abs(cx - t_cx) + jnp.abs(cy - t_cy)
              + jnp.abs(w - t_w) + jnp.abs(h - t_h))

    hw, hh = 0.5 * w, 0.5 * h
    x0, y0 = cx - hw, cy - hh
    x1, y1 = cx + hw, cy + hh
    thw, thh = 0.5 * t_w, 0.5 * t_h
    tx0, ty0 = t_cx - thw, t_cy - thh
    tx1, ty1 = t_cx + thw, t_cy + thh
    area1 = (x1 - x0) * (y1 - y0)
    area2 = (tx1 - tx0) * (ty1 - ty0)
    iw = jnp.clip(jnp.minimum(x1, tx1) - jnp.maximum(x0, tx0), 0.0)
    ih = jnp.clip(jnp.minimum(y1, ty1) - jnp.maximum(y0, ty0), 0.0)
    inter = iw * ih
    union = area1 + area2 - inter
    iou = inter / union
    ew = jnp.maximum(x1, tx1) - jnp.minimum(x0, tx0)
    eh = jnp.maximum(y1, ty1) - jnp.minimum(y0, ty0)
    earea = ew * eh
    c_giou = -(iou - (earea - union) / earea)

    img = dm_ref[0]
    dn = img / (jnp.max(img) + 1e-06)
    gx = cx * 2.0 - 1.0
    gy = cy * 2.0 - 1.0
    ix = ((gx + 1.0) * W - 1.0) / 2.0
    iy = ((gy + 1.0) * H - 1.0) / 2.0
    col_iota = jax.lax.broadcasted_iota(jnp.int32, (Q, W), 1).astype(jnp.float32)
    wmat_y = jnp.maximum(1.0 - jnp.abs(col_iota - iy), 0.0)
    wmat_x = jnp.maximum(1.0 - jnp.abs(col_iota - ix), 0.0)
    rows = jnp.dot(wmat_y, dn, preferred_element_type=jnp.float32,
                   precision=jax.lax.Precision.HIGHEST)
    sampled = jnp.sum(rows * wmat_x, axis=1, keepdims=True)

    C = (COST_BBOX * c_bbox + COST_CLASS * c_class
         + COST_GIOU * c_giou + COST_DENSITY * (-sampled))

    areas = tbT_ref[b, 2:3, :] * tbT_ref[b, 3:4, :]
    kvec = jnp.where(areas < 0.005, 6, 2)
    q_iota = jax.lax.broadcasted_iota(jnp.int32, (Q, G), 0)
    Cw = C
    mm = jnp.zeros((Q, G), dtype=jnp.float32)
    for s in range(6):
        mval = jnp.min(Cw, axis=0, keepdims=True)
        idx = jnp.min(jnp.where(Cw == mval, q_iota, Q), axis=0,
                      keepdims=True)
        sel = q_iota == idx
        mm = jnp.maximum(mm, jnp.where(jnp.logical_and(sel, s < kvec),
                                       1.0, 0.0))
        Cw = jnp.where(sel, jnp.inf, Cw)

    nmatch = jnp.sum(mm, axis=1, keepdims=True)
    conflict = nmatch > 1.5
    matched = mm > 0.5
    masked_C = jnp.where(matched, C, jnp.inf)
    mrow = jnp.min(masked_C, axis=1, keepdims=True)
    g_iota = jax.lax.broadcasted_iota(jnp.int32, (Q, G), 1)
    bidx = jnp.min(jnp.where(masked_C == mrow, g_iota, G), axis=1,
                   keepdims=True)
    mm = jnp.where(conflict, jnp.where(g_iota == bidx, 1.0, 0.0), mm)

    match_ref[0] = mm > 0.5
    c_ref[0] = C


@jax.jit
def kernel(pred_logits, pred_boxes, tgt_labels, tgt_boxes, density_map):
    B, Q, NC = pred_logits.shape
    G = tgt_labels.shape[1]
    H, W = density_map.shape[2], density_map.shape[3]

    tbT = tgt_boxes.transpose(0, 2, 1)
    dm = density_map.reshape(B, H, W)

    matching, c_all = pl.pallas_call(
        _matcher_body,
        grid=(B,),
        in_specs=[
            pl.BlockSpec((1, Q, NC), lambda b: (b, 0, 0)),
            pl.BlockSpec((1, Q, 4), lambda b: (b, 0, 0)),
            pl.BlockSpec((B, G), lambda b: (0, 0)),
            pl.BlockSpec((B, 4, G), lambda b: (0, 0, 0)),
            pl.BlockSpec((1, H, W), lambda b: (b, 0, 0)),
        ],
        out_specs=[
            pl.BlockSpec((1, Q, G), lambda b: (b, 0, 0)),
            pl.BlockSpec((1, Q, G), lambda b: (b, 0, 0)),
        ],
        out_shape=[
            jax.ShapeDtypeStruct((B, Q, G), jnp.bool_),
            jax.ShapeDtypeStruct((B, Q, G), jnp.float32),
        ],
        compiler_params=pltpu.CompilerParams(
            dimension_semantics=("parallel",),
        ),
    )(pred_logits, pred_boxes, tgt_labels, tbT, dm)
    return matching, c_all
